# baseline (speedup 1.0000x reference)
import functools

import jax
import jax.numpy as jnp
from jax import lax
from jax.experimental import pallas as pl
from jax.experimental.pallas import tpu as pltpu
from jax.experimental.pallas import tpu_sc as plsc

_N = 10000
_R = 320
_G = _R // 16
_C = _R // 64
_NC = 2
_NS = 16
_MAGIC = 0x5F3759DF


def _iota16():
    return lax.broadcasted_iota(jnp.int32, (16,), 0)


def _compute_side(rand_v, tail_v, out_v):
    i16 = _iota16()
    c13 = jnp.full((16,), 13, jnp.int32)
    c15 = jnp.full((16,), 15, jnp.int32)
    for g in range(_G):
        idx0 = 3 * i16 + (g * 48)
        idx1 = idx0 + 1
        idx2 = idx0 + 2
        l0 = plsc.load_gather(rand_v, [idx0])
        l1 = plsc.load_gather(rand_v, [idx1])
        l2 = plsc.load_gather(rand_v, [idx2])
        s = l0 * l0 + l1 * l1 + l2 * l2
        s = jnp.maximum(s, 1e-24)
        i = plsc.bitcast(s, jnp.int32)
        i = _MAGIC - lax.shift_right_logical(i, 1)
        y = plsc.bitcast(i, jnp.float32)
        for _ in range(3):
            y = y * (1.5 - 0.5 * s * y * y)
        scale = 10.0 * y
        ridx = i16 + (g * 16)
        a = plsc.load_gather(tail_v, [ridx, c13])
        b = plsc.load_gather(tail_v, [ridx, c15])
        o0 = l0 * scale + jnp.where(a != 0.0, -10.0, 0.0)
        o1 = l1 * scale
        o2 = l2 * scale + jnp.where(b != 0.0, -10.0, 0.0)
        plsc.store_scatter(out_v, [idx0], o0)
        plsc.store_scatter(out_v, [idx1], o1)
        plsc.store_scatter(out_v, [idx2], o2)


def _body(xs_g, xt_g, lr, rr, left_o, right_o,
          idx_v, tail_l, tail_r, rand_l, rand_r, out_l, out_r,
          sem_l, sem_r, sem_rl, sem_rr):
    cid = lax.axis_index("c")
    sid = lax.axis_index("s")
    wid = sid * _NC + cid
    base = jnp.minimum(wid * _R, _N - _R)

    i16 = _iota16()
    for c in range(_C):
        for k in range(4):
            g = c * 4 + k
            idx_v[c, pl.ds(k * 16, 16)] = 8 * base + 7 + 128 * g + 8 * i16

    cp_l = []
    cp_r = []
    for c in range(_C):
        cp_l.append(pltpu.async_copy(
            xs_g.at[idx_v.at[c]], tail_l.at[pl.ds(c * 64, 64)], sem_l))
        cp_r.append(pltpu.async_copy(
            xt_g.at[idx_v.at[c]], tail_r.at[pl.ds(c * 64, 64)], sem_r))
    crl = pltpu.async_copy(lr.at[pl.ds(base * 3, 3 * _R)], rand_l, sem_rl)
    crr = pltpu.async_copy(rr.at[pl.ds(base * 3, 3 * _R)], rand_r, sem_rr)

    crl.wait()
    for cp in cp_l:
        cp.wait()
    _compute_side(rand_l, tail_l, out_l)
    pltpu.sync_copy(out_l, left_o.at[pl.ds(base * 3, 3 * _R)])

    crr.wait()
    for cp in cp_r:
        cp.wait()
    _compute_side(rand_r, tail_r, out_r)
    pltpu.sync_copy(out_r, right_o.at[pl.ds(base * 3, 3 * _R)])


@functools.partial(jax.jit, static_argnames=())
def _run(xs_g, xt_g, lr, rr):
    f32 = jnp.float32
    k = functools.partial(
        pl.kernel,
        out_type=(jax.ShapeDtypeStruct((3 * _N,), f32),
                  jax.ShapeDtypeStruct((3 * _N,), f32)),
        mesh=plsc.VectorSubcoreMesh(core_axis_name="c", subcore_axis_name="s"),
        compiler_params=pltpu.CompilerParams(
            needs_layout_passes=False, use_tc_tiling_on_sc=False),
        scratch_types=[
            pltpu.VMEM((_C, 64), jnp.int32),
            pltpu.VMEM((_R, 16), f32),
            pltpu.VMEM((_R, 16), f32),
            pltpu.VMEM((3 * _R,), f32),
            pltpu.VMEM((3 * _R,), f32),
            pltpu.VMEM((3 * _R,), f32),
            pltpu.VMEM((3 * _R,), f32),
            pltpu.SemaphoreType.DMA,
            pltpu.SemaphoreType.DMA,
            pltpu.SemaphoreType.DMA,
            pltpu.SemaphoreType.DMA,
        ],
    )(_body)
    return k(xs_g, xt_g, lr, rr)


def kernel(x_s, x_t, edge_index, left_rand, right_rand):
    del edge_index
    n, d = x_s.shape
    xs_g = x_s.reshape(n * d // 16, 16)
    xt_g = x_t.reshape(n * d // 16, 16)
    lr = left_rand.reshape(-1)
    rr = right_rand.reshape(-1)
    left, right = _run(xs_g, xt_g, lr, rr)
    return left.reshape(n, 3), right.reshape(n, 3)


# speedup vs baseline: 1.0012x; 1.0012x over previous
import functools

import jax
import jax.numpy as jnp
from jax import lax
from jax.experimental import pallas as pl
from jax.experimental.pallas import tpu as pltpu
from jax.experimental.pallas import tpu_sc as plsc

_N = 10000
_R = 320
_G = _R // 16
_C = _R // 64
_NC = 2
_NS = 16
_MAGIC = 0x5F3759DF


def _iota16():
    return lax.broadcasted_iota(jnp.int32, (16,), 0)


def _compute_side(rand_v, tail_v, out_v):
    i16 = _iota16()
    c13 = jnp.full((16,), 13, jnp.int32)
    c15 = jnp.full((16,), 15, jnp.int32)
    for g in range(_G):
        idx0 = 3 * i16 + (g * 48)
        idx1 = idx0 + 1
        idx2 = idx0 + 2
        l0 = plsc.load_gather(rand_v, [idx0])
        l1 = plsc.load_gather(rand_v, [idx1])
        l2 = plsc.load_gather(rand_v, [idx2])
        s = l0 * l0 + l1 * l1 + l2 * l2
        s = jnp.maximum(s, 1e-24)
        i = plsc.bitcast(s, jnp.int32)
        i = _MAGIC - lax.shift_right_logical(i, 1)
        y = plsc.bitcast(i, jnp.float32)
        for _ in range(3):
            y = y * (1.5 - 0.5 * s * y * y)
        scale = 10.0 * y
        ridx = i16 + (g * 16)
        a = plsc.load_gather(tail_v, [ridx, c13])
        b = plsc.load_gather(tail_v, [ridx, c15])
        o0 = l0 * scale + jnp.where(a != 0.0, -10.0, 0.0)
        o1 = l1 * scale
        o2 = l2 * scale + jnp.where(b != 0.0, -10.0, 0.0)
        plsc.store_scatter(out_v, [idx0], o0)
        plsc.store_scatter(out_v, [idx1], o1)
        plsc.store_scatter(out_v, [idx2], o2)


def _body(xs_g, xt_g, lr, rr, left_o, right_o,
          idx_v, tail_l, tail_r, rand_l, rand_r, out_l, out_r,
          sem_l, sem_r, sem_rl, sem_rr):
    cid = lax.axis_index("c")
    sid = lax.axis_index("s")
    wid = sid * _NC + cid
    base = jnp.minimum(wid * _R, _N - _R)

    i16 = _iota16()
    for c in range(_C):
        for k in range(4):
            g = c * 4 + k
            idx_v[c, pl.ds(k * 16, 16)] = 8 * base + 7 + 128 * g + 8 * i16

    cp_l = []
    cp_r = []
    for c in range(_C):
        cp_l.append(pltpu.async_copy(
            xs_g.at[idx_v.at[c]], tail_l.at[pl.ds(c * 64, 64)], sem_l))
        cp_r.append(pltpu.async_copy(
            xt_g.at[idx_v.at[c]], tail_r.at[pl.ds(c * 64, 64)], sem_r))
    crl = pltpu.async_copy(lr.at[pl.ds(base * 3, 3 * _R)], rand_l, sem_rl)
    crr = pltpu.async_copy(rr.at[pl.ds(base * 3, 3 * _R)], rand_r, sem_rr)

    crl.wait()
    for cp in cp_l:
        cp.wait()
    _compute_side(rand_l, tail_l, out_l)
    pltpu.sync_copy(out_l, left_o.at[pl.ds(base * 3, 3 * _R)])

    crr.wait()
    for cp in cp_r:
        cp.wait()
    _compute_side(rand_r, tail_r, out_r)
    pltpu.sync_copy(out_r, right_o.at[pl.ds(base * 3, 3 * _R)])


@functools.partial(jax.jit, static_argnames=())
def _run(xs_g, xt_g, lr, rr):
    f32 = jnp.float32
    k = functools.partial(
        pl.kernel,
        out_type=(jax.ShapeDtypeStruct((3 * _N,), f32),
                  jax.ShapeDtypeStruct((3 * _N,), f32)),
        mesh=plsc.VectorSubcoreMesh(core_axis_name="c", subcore_axis_name="s"),
        compiler_params=pltpu.CompilerParams(
            needs_layout_passes=False, use_tc_tiling_on_sc=False,
            skip_device_barrier=True),
        scratch_types=[
            pltpu.VMEM((_C, 64), jnp.int32),
            pltpu.VMEM((_R, 16), f32),
            pltpu.VMEM((_R, 16), f32),
            pltpu.VMEM((3 * _R,), f32),
            pltpu.VMEM((3 * _R,), f32),
            pltpu.VMEM((3 * _R,), f32),
            pltpu.VMEM((3 * _R,), f32),
            pltpu.SemaphoreType.DMA,
            pltpu.SemaphoreType.DMA,
            pltpu.SemaphoreType.DMA,
            pltpu.SemaphoreType.DMA,
        ],
    )(_body)
    return k(xs_g, xt_g, lr, rr)


def kernel(x_s, x_t, edge_index, left_rand, right_rand):
    del edge_index
    n, d = x_s.shape
    xs_g = x_s.reshape(n * d // 16, 16)
    xt_g = x_t.reshape(n * d // 16, 16)
    lr = left_rand.reshape(-1)
    rr = right_rand.reshape(-1)
    left, right = _run(xs_g, xt_g, lr, rr)
    return left.reshape(n, 3), right.reshape(n, 3)
